# Initial kernel scaffold; baseline (speedup 1.0000x reference)
import jax
import jax.numpy as jnp
from jax.experimental import pallas as pl


def kernel(x, edge_index):
    raise NotImplementedError("write your pallas kernel here")


# speedup vs baseline: 14.7254x; 14.7254x over previous
import functools

import jax
import jax.numpy as jnp
from jax import lax
from jax.experimental import pallas as pl
from jax.experimental.pallas import tpu as pltpu
from jax.experimental.pallas import tpu_sc as plsc

N = 10000
E = 160000
D = 256
H = 128
NC, NS, L = 2, 16, 16
NPAD = 10240

K1 = 40
C1 = (E // (NC * NS)) // K1
K3 = 80
C3 = (E // NS) // K3

NPZ = 10112
TPT = NPZ // NS

_mesh = plsc.VectorSubcoreMesh(
    core_axis_name="c", subcore_axis_name="s", num_cores=NC, num_subcores=NS)


def _fill_zeros_1d(ref, n):
    def body(i, _):
        ref[pl.ds(i * L, L)] = jnp.zeros((L,), jnp.float32)
        return 0
    lax.fori_loop(0, n // L, body, 0)


@functools.partial(
    pl.kernel,
    out_type=(jax.ShapeDtypeStruct((NPAD,), jnp.float32),
              jax.ShapeDtypeStruct((NPAD,), jnp.float32)),
    mesh=_mesh,
    scratch_types=(
        pltpu.VMEM((C1, K1), jnp.int32),
        pltpu.VMEM((48,), jnp.float32),
        pltpu.VMEM((640,), jnp.float32),
        pltpu.VMEM_SHARED((NPAD,), jnp.float32),
    ),
)
def _deg_kernel(col_hbm, d0_hbm, d1_hbm, idx_v, ones_v, buf_v, deg_sh):
    c = lax.axis_index("c")
    s = lax.axis_index("s")
    w = c * NS + s

    def fill_ones(i, _):
        ones_v[pl.ds(i * L, L)] = jnp.ones((L,), jnp.float32)
        return 0
    lax.fori_loop(0, 48 // L, fill_ones, 0)
    _fill_zeros_1d(buf_v, 640)
    pltpu.sync_copy(buf_v, deg_sh.at[pl.ds(s * 640, 640)])
    plsc.subcore_barrier()

    pltpu.sync_copy(col_hbm.at[w], idx_v)

    def step(j, _):
        pltpu.sync_copy(ones_v.at[pl.ds(0, K1)], deg_sh.at[idx_v.at[j]],
                        add=True)
        return 0
    lax.fori_loop(0, C1, step, 0)
    plsc.subcore_barrier()

    pltpu.sync_copy(deg_sh.at[pl.ds(s * 640, 640)], buf_v)

    @pl.when(c == 0)
    def _():
        pltpu.sync_copy(buf_v, d0_hbm.at[pl.ds(s * 640, 640)])

    @pl.when(c == 1)
    def _():
        pltpu.sync_copy(buf_v, d1_hbm.at[pl.ds(s * 640, 640)])


def _scale_body(d0_ref, d1_ref, x_ref, y0_ref, y1_ref):
    deg = d0_ref[...] + d1_ref[...] + 1.0
    dinv = lax.rsqrt(deg)
    y = x_ref[...] * dinv
    y0_ref[...] = y[:, :H]
    y1_ref[...] = y[:, H:]


_R2 = 2000

_scale_call = pl.pallas_call(
    _scale_body,
    grid=(N // _R2,),
    in_specs=[
        pl.BlockSpec((_R2, 1), lambda i: (i, 0)),
        pl.BlockSpec((_R2, 1), lambda i: (i, 0)),
        pl.BlockSpec((_R2, D), lambda i: (i, 0)),
    ],
    out_specs=[
        pl.BlockSpec((_R2, H), lambda i: (i, 0)),
        pl.BlockSpec((_R2, H), lambda i: (i, 0)),
    ],
    out_shape=[jax.ShapeDtypeStruct((N, H), jnp.float32),
               jax.ShapeDtypeStruct((N, H), jnp.float32)],
)


@functools.partial(
    pl.kernel,
    out_type=(jax.ShapeDtypeStruct((NPZ, H), jnp.float32),
              jax.ShapeDtypeStruct((NPZ, H), jnp.float32)),
    mesh=_mesh,
    scratch_types=(
        pltpu.VMEM((C3, K3), jnp.int32),
        pltpu.VMEM((C3, K3), jnp.int32),
        pltpu.VMEM((K3, H), jnp.float32),
        pltpu.VMEM_SHARED((NPZ, H), jnp.float32),
        pltpu.SemaphoreType.DMA,
    ),
)
def _edge_kernel(row_hbm, col_hbm, y0_hbm, y1_hbm, z0_hbm, z1_hbm,
                 idxr_v, idxc_v, gbuf_v, z_sh, sem):
    c = lax.axis_index("c")
    s = lax.axis_index("s")

    def zfill(r, _):
        def zfill_c(k, _):
            gbuf_v[r, pl.ds(k * L, L)] = jnp.zeros((L,), jnp.float32)
            return 0
        lax.fori_loop(0, H // L, zfill_c, 0)
        return 0
    lax.fori_loop(0, K3, zfill, 0)

    def zinit(j, _):
        pltpu.sync_copy(gbuf_v, z_sh.at[pl.ds(s * TPT + j * K3, K3), :])
        return 0
    lax.fori_loop(0, TPT // K3, zinit, 0)
    pltpu.sync_copy(gbuf_v.at[pl.ds(0, TPT % K3)],
                    z_sh.at[pl.ds(s * TPT + (TPT // K3) * K3, TPT % K3), :])
    plsc.subcore_barrier()

    pltpu.sync_copy(row_hbm.at[s], idxr_v)
    pltpu.sync_copy(col_hbm.at[s], idxc_v)

    def run(y_hbm, z_hbm):
        def step(j, _):
            pltpu.async_copy(y_hbm.at[idxr_v.at[j]], gbuf_v, sem).wait()
            pltpu.sync_copy(gbuf_v, z_sh.at[idxc_v.at[j]], add=True)
            return 0
        lax.fori_loop(0, C3, step, 0)
        plsc.subcore_barrier()

        def drain_chunk(r0, nr):
            pltpu.sync_copy(z_sh.at[pl.ds(s * TPT + r0, nr), :],
                            gbuf_v.at[pl.ds(0, nr)])
            pltpu.sync_copy(gbuf_v.at[pl.ds(0, nr)],
                            z_hbm.at[pl.ds(s * TPT + r0, nr), :])

        def drain(j, _):
            drain_chunk(j * K3, K3)
            return 0
        lax.fori_loop(0, TPT // K3, drain, 0)
        drain_chunk((TPT // K3) * K3, TPT % K3)

    @pl.when(c == 0)
    def _():
        run(y0_hbm, z0_hbm)

    @pl.when(c == 1)
    def _():
        run(y1_hbm, z1_hbm)


def _final_body(d0_ref, d1_ref, x_ref, z0_ref, z1_ref, o_ref):
    deg = d0_ref[...] + d1_ref[...] + 1.0
    dinv = lax.rsqrt(deg)
    x = x_ref[...]
    z = jnp.concatenate([z0_ref[...], z1_ref[...]], axis=1)
    o_ref[...] = 0.5 * (x + dinv * z + (dinv * dinv) * x)


_final_call = pl.pallas_call(
    _final_body,
    grid=(N // _R2,),
    in_specs=[
        pl.BlockSpec((_R2, 1), lambda i: (i, 0)),
        pl.BlockSpec((_R2, 1), lambda i: (i, 0)),
        pl.BlockSpec((_R2, D), lambda i: (i, 0)),
        pl.BlockSpec((_R2, H), lambda i: (i, 0)),
        pl.BlockSpec((_R2, H), lambda i: (i, 0)),
    ],
    out_specs=pl.BlockSpec((_R2, D), lambda i: (i, 0)),
    out_shape=jax.ShapeDtypeStruct((N, D), jnp.float32),
)


def kernel(x, edge_index):
    row = edge_index[0]
    col = edge_index[1]
    d0, d1 = _deg_kernel(col.reshape(NC * NS, C1, K1))
    d0c = d0[:N, None]
    d1c = d1[:N, None]
    y0, y1 = _scale_call(d0c, d1c, x)
    z0, z1 = _edge_kernel(row.reshape(NS, C3, K3), col.reshape(NS, C3, K3),
                          y0, y1)
    return _final_call(d0c, d1c, x, z0[:N], z1[:N])
